# baseline (speedup 1.0000x reference)
import jax
import jax.numpy as jnp
from jax.experimental import pallas as pl
from jax.experimental.pallas import tpu as pltpu

_R = 128
_N = 100000
_C = 12544
_G = 8


def _argmax_kernel(x_ref, out_ref, vmax_ref, vidx_ref):
    j = pl.program_id(0)
    x = x_ref[...]
    lanes = jax.lax.broadcasted_iota(jnp.int32, (_R, _C), 1)
    x = jnp.where(jnp.logical_or(j < _G - 1, lanes < _N - (_G - 1) * _C), x, -1.0)
    lmax = jnp.max(x, axis=1, keepdims=True)
    lidx = (jnp.argmax(x, axis=1).astype(jnp.int32) + j * _C)[:, None]

    @pl.when(j == 0)
    def _init():
        vmax_ref[...] = lmax
        vidx_ref[...] = lidx

    @pl.when(j > 0)
    def _acc():
        better = lmax > vmax_ref[...]
        vmax_ref[...] = jnp.where(better, lmax, vmax_ref[...])
        vidx_ref[...] = jnp.where(better, lidx, vidx_ref[...])

    @pl.when(j == _G - 1)
    def _fin():
        out_ref[...] = vidx_ref[...]


def kernel(probs):
    out = pl.pallas_call(
        _argmax_kernel,
        grid=(_G,),
        in_specs=[pl.BlockSpec((_R, _C), lambda j: (0, j))],
        out_specs=pl.BlockSpec((_R, 1), lambda j: (0, 0)),
        out_shape=jax.ShapeDtypeStruct((_R, 1), jnp.int32),
        scratch_shapes=[
            pltpu.VMEM((_R, 1), jnp.float32),
            pltpu.VMEM((_R, 1), jnp.int32),
        ],
    )(probs)
    return out[:, 0]


# speedup vs baseline: 1.0100x; 1.0100x over previous
import functools

import jax
import jax.numpy as jnp
from jax.experimental import pallas as pl
from jax.experimental.pallas import tpu as pltpu

_R = 128
_N = 100000
_C = 6272
_K = 4
_G = 4


def _argmax_kernel(*args):
    x_refs = args[:_K]
    out_ref, vmax_ref, vidx_ref = args[_K:]
    j = pl.program_id(0)

    best_v = None
    best_i = None
    for k in range(_K):
        x = x_refs[k][...]
        if k == _K - 1:
            bound = _N - ((_G - 1) * _K + k) * _C
            lanes = jax.lax.broadcasted_iota(jnp.int32, (_R, _C), 1)
            x = jnp.where(jnp.logical_or(j < _G - 1, lanes < bound), x, -1.0)
        lmax = jnp.max(x, axis=1, keepdims=True)
        lidx = jnp.argmax(x, axis=1).astype(jnp.int32)[:, None] + (j * _K + k) * _C
        if best_v is None:
            best_v, best_i = lmax, lidx
        else:
            upd = lmax > best_v
            best_i = jnp.where(upd, lidx, best_i)
            best_v = jnp.where(upd, lmax, best_v)

    @pl.when(j == 0)
    def _init():
        vmax_ref[...] = best_v
        vidx_ref[...] = best_i

    @pl.when(j > 0)
    def _acc():
        upd = best_v > vmax_ref[...]
        vmax_ref[...] = jnp.where(upd, best_v, vmax_ref[...])
        vidx_ref[...] = jnp.where(upd, best_i, vidx_ref[...])

    @pl.when(j == _G - 1)
    def _fin():
        out_ref[...] = vidx_ref[...]


def _imap(k, j):
    return (0, j * _K + k)


def kernel(probs):
    out = pl.pallas_call(
        _argmax_kernel,
        grid=(_G,),
        in_specs=[
            pl.BlockSpec((_R, _C), functools.partial(_imap, k)) for k in range(_K)
        ],
        out_specs=pl.BlockSpec((_R, 1), lambda j: (0, 0)),
        out_shape=jax.ShapeDtypeStruct((_R, 1), jnp.int32),
        scratch_shapes=[
            pltpu.VMEM((_R, 1), jnp.float32),
            pltpu.VMEM((_R, 1), jnp.int32),
        ],
    )(*([probs] * _K))
    return out[:, 0]
